# baseline (speedup 1.0000x reference)
import functools

import jax
import jax.numpy as jnp
from jax import lax
from jax.experimental import pallas as pl
from jax.experimental.pallas import tpu as pltpu
from jax.experimental.pallas import tpu_sc as plsc

_N = 10000
_D = 128
_E = 320000
_NSC = 2
_NTILE = 16
_C = 32
_NCH = 320
_SB = 16
_NSB = _NCH // _SB
_EPAD = _NSC * _NTILE * _NCH * _C
_NPAD = 8
_WR_TILES = 10
_ROWS_OUT = _N // _WR_TILES


def _sc_segment_sum(h, src2d, dst2d, zeros):
    mesh = plsc.VectorSubcoreMesh(
        core_axis_name="c", subcore_axis_name="s",
        num_cores=_NSC, num_subcores=_NTILE)

    @functools.partial(
        pl.kernel,
        out_type=jax.ShapeDtypeStruct((_NSC * _N, _D), jnp.float32),
        mesh=mesh,
        scratch_types=[
            pltpu.VMEM((1, _SB, _C), jnp.int32),
            pltpu.VMEM((1, _SB, _C), jnp.int32),
            pltpu.VMEM((_C, _D // 2), jnp.int32),
            pltpu.VMEM((_C, _D // 2), jnp.int32),
            pltpu.VMEM((_C, _D), jnp.float32),
            pltpu.VMEM_SHARED((_N + _NPAD, _D), jnp.float32),
            pltpu.VMEM_SHARED((_N, _D // 2), jnp.int32),
            pltpu.SemaphoreType.DMA,
            pltpu.SemaphoreType.DMA,
        ],
        compiler_params=pltpu.CompilerParams(use_tc_tiling_on_sc=False,
                                             needs_layout_passes=False),
    )
    def run(h_hbm, src_hbm, dst_hbm, z_hbm, out_hbm,
            src_v, dst_v, rows0, rows1, conv, acc, table, sem0, sem1):
        cid = lax.axis_index("c")
        sid = lax.axis_index("s")

        @pl.when(sid == 0)
        def _zero():
            pltpu.sync_copy(z_hbm, acc)

        @pl.when(sid == 1)
        def _stage_table():
            pltpu.sync_copy(h_hbm, table)

        tid = cid * _NTILE + sid
        plsc.subcore_barrier()

        rows = (rows0, rows1)
        sems = (sem0, sem1)

        def superblock(sb, carry):
            pltpu.sync_copy(src_hbm.at[pl.ds(tid, 1), pl.ds(sb * _SB, _SB)],
                            src_v)
            pltpu.sync_copy(dst_hbm.at[pl.ds(tid, 1), pl.ds(sb * _SB, _SB)],
                            dst_v)
            srcs = (h_hbm, table)
            pltpu.async_copy(srcs[0].at[src_v.at[0, 0]], rows0, sem0)
            pltpu.async_copy(srcs[1].at[src_v.at[0, 1]], rows1, sem1)

            def step(j, b):
                pltpu.make_async_copy(srcs[b].at[src_v.at[0, j]],
                                      rows[b], sems[b]).wait()

                def crow(r, c):
                    buf = rows[b]
                    for g in range(4):
                        w = buf[r, pl.ds(16 * g, 16)]
                        conv[r, pl.ds(32 * g, 16)] = plsc.bitcast(
                            w << 16, jnp.float32)
                        conv[r, pl.ds(32 * g + 16, 16)] = plsc.bitcast(
                            w & jnp.int32(-65536), jnp.float32)
                    return c

                lax.fori_loop(0, _C, crow, 0)
                pltpu.sync_copy(conv, acc.at[dst_v.at[0, j]], add=True)

                @pl.when(j + 2 < _SB)
                def _next():
                    pltpu.async_copy(srcs[b].at[src_v.at[0, j + 2]],
                                     rows[b], sems[b])

            def dbl(i, c):
                step(i * 2, 0)
                step(i * 2 + 1, 1)
                return c

            lax.fori_loop(0, _SB // 2, dbl, 0)
            return carry

        lax.fori_loop(0, _NSB, superblock, 0)
        plsc.subcore_barrier()

        @pl.when(sid < _WR_TILES)
        def _writeout():
            rbase = sid * _ROWS_OUT
            pltpu.sync_copy(acc.at[pl.ds(rbase, _ROWS_OUT)],
                            out_hbm.at[pl.ds(cid * _N + rbase, _ROWS_OUT)])

    return run(h, src2d, dst2d, zeros)


_B = 1000


def _pack_bf16(v):
    a = v.astype(jnp.bfloat16)
    parts = []
    for g in range(4):
        lo = lax.convert_element_type(
            lax.bitcast_convert_type(a[:, 32 * g:32 * g + 16], jnp.uint16),
            jnp.int32)
        hi = lax.convert_element_type(
            lax.bitcast_convert_type(a[:, 32 * g + 16:32 * g + 32], jnp.uint16),
            jnp.int32)
        parts.append((hi << 16) | lo)
    return jnp.concatenate(parts, axis=1)


def _reparam(v, escale):
    v0 = v[:, 0:1]
    time = escale / (1.0 + jnp.exp(-v0)) + 1.1
    sq = jnp.maximum(jnp.sum(v * v, axis=1, keepdims=True) - v0 * v0, 1e-8)
    sc = (time * time - 1.0) / sq
    scaled = v * jnp.sqrt(sc)
    col = lax.broadcasted_iota(jnp.int32, v.shape, 1)
    return jnp.where(col == 0, time, scaled)


def _centroid(s):
    s0 = s[:, 0:1]
    neg_inner = 2.0 * s0 * s0 - jnp.sum(s * s, axis=1, keepdims=True)
    return s / jnp.sqrt(jnp.maximum(jnp.abs(neg_inner), 1e-8))


def _tc1_body(x_ref, w1s_ref, w1t_ref, b1_ref, p_ref, out_ref):
    x = x_ref[...]
    sq = jnp.sum(x * x, axis=1, keepdims=True)
    nrm = jnp.sqrt(jnp.maximum(sq, 1e-8))
    e = jnp.exp(nrm)
    shn = 0.5 * (e - 1.0 / e)
    xs = x * (shn / nrm)
    time = jnp.sqrt(1.0 + jnp.sum(xs * xs, axis=1, keepdims=True))
    v = (jnp.dot(xs, w1s_ref[...], preferred_element_type=jnp.float32,
                 precision=lax.Precision.HIGHEST)
         + time * w1t_ref[...] + b1_ref[...])
    out_ref[...] = _pack_bf16(_reparam(v, p_ref[0, 0]))


def _tc2_body(lo_ref, hi_ref, w2_ref, b2_ref, p_ref, h_ref, y_ref):
    s = lo_ref[...] + hi_ref[...]
    h = _centroid(s)
    h_ref[...] = h
    xr = jnp.maximum(h, 0.0)
    v = jnp.dot(xr, w2_ref[...], preferred_element_type=jnp.float32,
                precision=lax.Precision.HIGHEST) + b2_ref[...]
    y_ref[...] = _pack_bf16(_reparam(v, p_ref[0, 1]))


def _tc3_body(lo_ref, hi_ref, h_ref, p_ref, out_ref):
    y = _centroid(lo_ref[...] + hi_ref[...])
    z = p_ref[0, 2] * h_ref[...] + p_ref[0, 3] * y
    out_ref[...] = _centroid(z)


def _row_spec():
    return pl.BlockSpec((_B, _D), lambda i: (i, 0))


def _hi_spec():
    return pl.BlockSpec((_B, _D), lambda i: (i + _N // _B, 0))


def _w_spec(shape):
    return pl.BlockSpec(shape, lambda i: (0, 0))


def _p_spec():
    return pl.BlockSpec(memory_space=pltpu.SMEM)


_GRID = (_N // _B,)


def _pk_spec():
    return pl.BlockSpec((_B, _D // 2), lambda i: (i, 0))


def _tc1(x, w1s, w1t, b1, params):
    return pl.pallas_call(
        _tc1_body,
        grid=_GRID,
        in_specs=[_row_spec(), _w_spec((_D, _D)), _w_spec((1, _D)),
                  _w_spec((1, _D)), _p_spec()],
        out_specs=_pk_spec(),
        out_shape=jax.ShapeDtypeStruct((_N, _D // 2), jnp.int32),
    )(x, w1s, w1t, b1, params)


def _tc2(partials, w2, b2, params):
    return pl.pallas_call(
        _tc2_body,
        grid=_GRID,
        in_specs=[_row_spec(), _hi_spec(), _w_spec((_D, _D)),
                  _w_spec((1, _D)), _p_spec()],
        out_specs=[_row_spec(), _pk_spec()],
        out_shape=[jax.ShapeDtypeStruct((_N, _D), jnp.float32),
                   jax.ShapeDtypeStruct((_N, _D // 2), jnp.int32)],
    )(partials, partials, w2, b2, params)


def _tc3(partials, h, params):
    return pl.pallas_call(
        _tc3_body,
        grid=_GRID,
        in_specs=[_row_spec(), _hi_spec(), _row_spec(), _p_spec()],
        out_specs=_row_spec(),
        out_shape=jax.ShapeDtypeStruct((_N, _D), jnp.float32),
    )(partials, partials, h, params)


def kernel(x, adj, W1, b1, scale1, W2, b2, scale2, wx, wy):
    npad = _EPAD - _E
    src2d = jnp.concatenate(
        [adj[0], jnp.zeros((npad,), jnp.int32)]).reshape(
            _NSC * _NTILE, _NCH, _C)
    dst2d = jnp.concatenate(
        [adj[1], _N + (jnp.arange(npad, dtype=jnp.int32) % _NPAD)]).reshape(
            _NSC * _NTILE, _NCH, _C)
    zeros = jnp.zeros((_N + _NPAD, _D), dtype=jnp.float32)
    params = jnp.stack([jnp.exp(scale1), jnp.exp(scale2), wx, wy]).reshape(1, 4)
    b1r = b1.reshape(1, _D)
    b2r = b2.reshape(1, _D)
    w1t = W1[0:1, :]
    w1s = W1[1:, :]

    h1_packed = _tc1(x, w1s, w1t, b1r, params)
    partials1 = _sc_segment_sum(h1_packed, src2d, dst2d, zeros)
    h, y_packed = _tc2(partials1, W2, b2r, params)
    partials2 = _sc_segment_sum(y_packed, src2d, dst2d, zeros)
    return _tc3(partials2, h, params)


# speedup vs baseline: 1.1470x; 1.1470x over previous
import functools

import jax
import jax.numpy as jnp
from jax import lax
from jax.experimental import pallas as pl
from jax.experimental.pallas import tpu as pltpu
from jax.experimental.pallas import tpu_sc as plsc

_N = 10000
_D = 128
_E = 320000
_NSC = 2
_NTILE = 16
_C = 128
_NCH = 80
_SB = 16
_NSB = _NCH // _SB
_EPAD = _NSC * _NTILE * _NCH * _C
_NPAD = 8
_WR_TILES = 10
_ROWS_OUT = _N // _WR_TILES


def _sc_segment_sum(h, src2d, dst2d, zeros):
    mesh = plsc.VectorSubcoreMesh(
        core_axis_name="c", subcore_axis_name="s",
        num_cores=_NSC, num_subcores=_NTILE)

    @functools.partial(
        pl.kernel,
        out_type=jax.ShapeDtypeStruct((_NSC * _N, _D), jnp.float32),
        mesh=mesh,
        scratch_types=[
            pltpu.VMEM((1, _SB, _C), jnp.int32),
            pltpu.VMEM((1, _SB, _C), jnp.int32),
            pltpu.VMEM((_C, _D // 2), jnp.int32),
            pltpu.VMEM((_C, _D // 2), jnp.int32),
            pltpu.VMEM((_C, _D), jnp.float32),
            pltpu.VMEM_SHARED((_N + _NPAD, _D), jnp.float32),
            pltpu.SemaphoreType.DMA,
            pltpu.SemaphoreType.DMA,
        ],
        compiler_params=pltpu.CompilerParams(use_tc_tiling_on_sc=False,
                                             needs_layout_passes=False),
    )
    def run(h_hbm, src_hbm, dst_hbm, z_hbm, out_hbm,
            src_v, dst_v, rows0, rows1, conv, acc, sem0, sem1):
        cid = lax.axis_index("c")
        sid = lax.axis_index("s")

        @pl.when(sid == 0)
        def _zero():
            pltpu.sync_copy(z_hbm, acc)

        tid = cid * _NTILE + sid
        plsc.subcore_barrier()

        rows = (rows0, rows1)
        sems = (sem0, sem1)

        def superblock(sb, carry):
            pltpu.sync_copy(src_hbm.at[pl.ds(tid, 1), pl.ds(sb * _SB, _SB)],
                            src_v)
            pltpu.sync_copy(dst_hbm.at[pl.ds(tid, 1), pl.ds(sb * _SB, _SB)],
                            dst_v)
            pltpu.async_copy(h_hbm.at[src_v.at[0, 0]], rows0, sem0)
            pltpu.async_copy(h_hbm.at[src_v.at[0, 1]], rows1, sem1)

            def step(j, b):
                pltpu.make_async_copy(h_hbm.at[src_v.at[0, j]],
                                      rows[b], sems[b]).wait()

                def crow(r, c):
                    buf = rows[b]
                    for g in range(4):
                        w = buf[r, pl.ds(16 * g, 16)]
                        conv[r, pl.ds(32 * g, 16)] = plsc.bitcast(
                            w << 16, jnp.float32)
                        conv[r, pl.ds(32 * g + 16, 16)] = plsc.bitcast(
                            w & jnp.int32(-65536), jnp.float32)
                    return c

                lax.fori_loop(0, _C, crow, 0)
                pltpu.sync_copy(conv, acc.at[dst_v.at[0, j]], add=True)

                @pl.when(j + 2 < _SB)
                def _next():
                    pltpu.async_copy(h_hbm.at[src_v.at[0, j + 2]],
                                     rows[b], sems[b])

            def dbl(i, c):
                step(i * 2, 0)
                step(i * 2 + 1, 1)
                return c

            lax.fori_loop(0, _SB // 2, dbl, 0)
            return carry

        lax.fori_loop(0, _NSB, superblock, 0)
        plsc.subcore_barrier()

        @pl.when(sid < _WR_TILES)
        def _writeout():
            rbase = sid * _ROWS_OUT
            pltpu.sync_copy(acc.at[pl.ds(rbase, _ROWS_OUT)],
                            out_hbm.at[pl.ds(cid * _N + rbase, _ROWS_OUT)])

    return run(h, src2d, dst2d, zeros)


_B = 1000


def _pack_bf16(v):
    a = v.astype(jnp.bfloat16)
    parts = []
    for g in range(4):
        lo = lax.convert_element_type(
            lax.bitcast_convert_type(a[:, 32 * g:32 * g + 16], jnp.uint16),
            jnp.int32)
        hi = lax.convert_element_type(
            lax.bitcast_convert_type(a[:, 32 * g + 16:32 * g + 32], jnp.uint16),
            jnp.int32)
        parts.append((hi << 16) | lo)
    return jnp.concatenate(parts, axis=1)


def _reparam(v, escale):
    v0 = v[:, 0:1]
    time = escale / (1.0 + jnp.exp(-v0)) + 1.1
    sq = jnp.maximum(jnp.sum(v * v, axis=1, keepdims=True) - v0 * v0, 1e-8)
    sc = (time * time - 1.0) / sq
    scaled = v * jnp.sqrt(sc)
    col = lax.broadcasted_iota(jnp.int32, v.shape, 1)
    return jnp.where(col == 0, time, scaled)


def _centroid(s):
    s0 = s[:, 0:1]
    neg_inner = 2.0 * s0 * s0 - jnp.sum(s * s, axis=1, keepdims=True)
    return s / jnp.sqrt(jnp.maximum(jnp.abs(neg_inner), 1e-8))


def _tc1_body(x_ref, w1s_ref, w1t_ref, b1_ref, p_ref, out_ref):
    x = x_ref[...]
    sq = jnp.sum(x * x, axis=1, keepdims=True)
    nrm = jnp.sqrt(jnp.maximum(sq, 1e-8))
    e = jnp.exp(nrm)
    shn = 0.5 * (e - 1.0 / e)
    xs = x * (shn / nrm)
    time = jnp.sqrt(1.0 + jnp.sum(xs * xs, axis=1, keepdims=True))
    v = (jnp.dot(xs, w1s_ref[...], preferred_element_type=jnp.float32,
                 precision=lax.Precision.HIGHEST)
         + time * w1t_ref[...] + b1_ref[...])
    out_ref[...] = _pack_bf16(_reparam(v, p_ref[0, 0]))


def _tc2_body(lo_ref, hi_ref, w2_ref, b2_ref, p_ref, h_ref, y_ref):
    s = lo_ref[...] + hi_ref[...]
    h = _centroid(s)
    h_ref[...] = h
    xr = jnp.maximum(h, 0.0)
    v = jnp.dot(xr, w2_ref[...], preferred_element_type=jnp.float32,
                precision=lax.Precision.HIGHEST) + b2_ref[...]
    y_ref[...] = _pack_bf16(_reparam(v, p_ref[0, 1]))


def _tc3_body(lo_ref, hi_ref, h_ref, p_ref, out_ref):
    y = _centroid(lo_ref[...] + hi_ref[...])
    z = p_ref[0, 2] * h_ref[...] + p_ref[0, 3] * y
    out_ref[...] = _centroid(z)


def _row_spec():
    return pl.BlockSpec((_B, _D), lambda i: (i, 0))


def _hi_spec():
    return pl.BlockSpec((_B, _D), lambda i: (i + _N // _B, 0))


def _w_spec(shape):
    return pl.BlockSpec(shape, lambda i: (0, 0))


def _p_spec():
    return pl.BlockSpec(memory_space=pltpu.SMEM)


_GRID = (_N // _B,)


def _pk_spec():
    return pl.BlockSpec((_B, _D // 2), lambda i: (i, 0))


def _tc1(x, w1s, w1t, b1, params):
    return pl.pallas_call(
        _tc1_body,
        grid=_GRID,
        in_specs=[_row_spec(), _w_spec((_D, _D)), _w_spec((1, _D)),
                  _w_spec((1, _D)), _p_spec()],
        out_specs=_pk_spec(),
        out_shape=jax.ShapeDtypeStruct((_N, _D // 2), jnp.int32),
    )(x, w1s, w1t, b1, params)


def _tc2(partials, w2, b2, params):
    return pl.pallas_call(
        _tc2_body,
        grid=_GRID,
        in_specs=[_row_spec(), _hi_spec(), _w_spec((_D, _D)),
                  _w_spec((1, _D)), _p_spec()],
        out_specs=[_row_spec(), _pk_spec()],
        out_shape=[jax.ShapeDtypeStruct((_N, _D), jnp.float32),
                   jax.ShapeDtypeStruct((_N, _D // 2), jnp.int32)],
    )(partials, partials, w2, b2, params)


def _tc3(partials, h, params):
    return pl.pallas_call(
        _tc3_body,
        grid=_GRID,
        in_specs=[_row_spec(), _hi_spec(), _row_spec(), _p_spec()],
        out_specs=_row_spec(),
        out_shape=jax.ShapeDtypeStruct((_N, _D), jnp.float32),
    )(partials, partials, h, params)


def kernel(x, adj, W1, b1, scale1, W2, b2, scale2, wx, wy):
    npad = _EPAD - _E
    src2d = jnp.concatenate(
        [adj[0], jnp.zeros((npad,), jnp.int32)]).reshape(
            _NSC * _NTILE, _NCH, _C)
    dst2d = jnp.concatenate(
        [adj[1], _N + (jnp.arange(npad, dtype=jnp.int32) % _NPAD)]).reshape(
            _NSC * _NTILE, _NCH, _C)
    zeros = jnp.zeros((_N + _NPAD, _D), dtype=jnp.float32)
    params = jnp.stack([jnp.exp(scale1), jnp.exp(scale2), wx, wy]).reshape(1, 4)
    b1r = b1.reshape(1, _D)
    b2r = b2.reshape(1, _D)
    w1t = W1[0:1, :]
    w1s = W1[1:, :]

    h1_packed = _tc1(x, w1s, w1t, b1r, params)
    partials1 = _sc_segment_sum(h1_packed, src2d, dst2d, zeros)
    h, y_packed = _tc2(partials1, W2, b2r, params)
    partials2 = _sc_segment_sum(y_packed, src2d, dst2d, zeros)
    return _tc3(partials2, h, params)


# speedup vs baseline: 1.2775x; 1.1138x over previous
import functools

import jax
import jax.numpy as jnp
from jax import lax
from jax.experimental import pallas as pl
from jax.experimental.pallas import tpu as pltpu
from jax.experimental.pallas import tpu_sc as plsc

_N = 10000
_D = 128
_E = 320000
_NSC = 2
_NTILE = 16
_C = 64
_NCH = 160
_EPAD = _NSC * _NTILE * _NCH * _C
_NPAD = 8
_WR_TILES = 10
_ROWS_OUT = _N // _WR_TILES


def _sc_segment_sum(h, src2d, dst2d, zeros):
    mesh = plsc.VectorSubcoreMesh(
        core_axis_name="c", subcore_axis_name="s",
        num_cores=_NSC, num_subcores=_NTILE)

    @functools.partial(
        pl.kernel,
        out_type=jax.ShapeDtypeStruct((_NSC * _N, _D), jnp.float32),
        mesh=mesh,
        scratch_types=[
            pltpu.VMEM((1, _NCH, _C), jnp.int32),
            pltpu.VMEM((1, _NCH, _C), jnp.int32),
            pltpu.VMEM((_C, _D // 2), jnp.int32),
            pltpu.VMEM((_C, _D // 2), jnp.int32),
            pltpu.VMEM((_C, _D), jnp.float32),
            pltpu.VMEM((_C, _D), jnp.float32),
            pltpu.VMEM_SHARED((_N + _NPAD, _D), jnp.float32),
            pltpu.SemaphoreType.DMA,
            pltpu.SemaphoreType.DMA,
            pltpu.SemaphoreType.DMA,
            pltpu.SemaphoreType.DMA,
        ],
        compiler_params=pltpu.CompilerParams(use_tc_tiling_on_sc=False,
                                             needs_layout_passes=False),
    )
    def run(h_hbm, src_hbm, dst_hbm, z_hbm, out_hbm,
            src_v, dst_v, rows0, rows1, conv0, conv1, acc,
            semg0, semg1, sems0, sems1):
        cid = lax.axis_index("c")
        sid = lax.axis_index("s")

        @pl.when(sid == 0)
        def _zero():
            pltpu.sync_copy(z_hbm, acc)

        tid = cid * _NTILE + sid
        pltpu.sync_copy(src_hbm.at[pl.ds(tid, 1)], src_v)
        pltpu.sync_copy(dst_hbm.at[pl.ds(tid, 1)], dst_v)
        plsc.subcore_barrier()

        rows = (rows0, rows1)
        convs = (conv0, conv1)
        semg = (semg0, semg1)
        sems = (sems0, sems1)

        pltpu.async_copy(h_hbm.at[src_v.at[0, 0]], rows0, semg0)
        pltpu.async_copy(h_hbm.at[src_v.at[0, 1]], rows1, semg1)

        def step(j, b):
            pltpu.make_async_copy(h_hbm.at[src_v.at[0, j]],
                                  rows[b], semg[b]).wait()

            @pl.when(j >= 2)
            def _drain():
                pltpu.make_async_copy(convs[b], acc.at[dst_v.at[0, 0]],
                                      sems[b]).wait()

            def crow(r, c):
                buf = rows[b]
                for g in range(4):
                    w = buf[r, pl.ds(16 * g, 16)]
                    convs[b][r, pl.ds(32 * g, 16)] = plsc.bitcast(
                        w << 16, jnp.float32)
                    convs[b][r, pl.ds(32 * g + 16, 16)] = plsc.bitcast(
                        w & jnp.int32(-65536), jnp.float32)
                return c

            lax.fori_loop(0, _C, crow, 0)
            pltpu.async_copy(convs[b], acc.at[dst_v.at[0, j]], sems[b],
                             add=True)

            @pl.when(j + 2 < _NCH)
            def _next():
                pltpu.async_copy(h_hbm.at[src_v.at[0, j + 2]],
                                 rows[b], semg[b])

        def dbl(i, c):
            step(i * 2, 0)
            step(i * 2 + 1, 1)
            return c

        lax.fori_loop(0, _NCH // 2, dbl, 0)
        for b in range(2):
            pltpu.make_async_copy(convs[b], acc.at[dst_v.at[0, 0]],
                                  sems[b]).wait()
        plsc.subcore_barrier()

        @pl.when(sid < _WR_TILES)
        def _writeout():
            rbase = sid * _ROWS_OUT
            pltpu.sync_copy(acc.at[pl.ds(rbase, _ROWS_OUT)],
                            out_hbm.at[pl.ds(cid * _N + rbase, _ROWS_OUT)])

    return run(h, src2d, dst2d, zeros)


_B = 1000


def _pack_bf16(v):
    a = v.astype(jnp.bfloat16)
    parts = []
    for g in range(4):
        lo = lax.convert_element_type(
            lax.bitcast_convert_type(a[:, 32 * g:32 * g + 16], jnp.uint16),
            jnp.int32)
        hi = lax.convert_element_type(
            lax.bitcast_convert_type(a[:, 32 * g + 16:32 * g + 32], jnp.uint16),
            jnp.int32)
        parts.append((hi << 16) | lo)
    return jnp.concatenate(parts, axis=1)


def _reparam(v, escale):
    v0 = v[:, 0:1]
    time = escale / (1.0 + jnp.exp(-v0)) + 1.1
    sq = jnp.maximum(jnp.sum(v * v, axis=1, keepdims=True) - v0 * v0, 1e-8)
    sc = (time * time - 1.0) / sq
    scaled = v * jnp.sqrt(sc)
    col = lax.broadcasted_iota(jnp.int32, v.shape, 1)
    return jnp.where(col == 0, time, scaled)


def _centroid(s):
    s0 = s[:, 0:1]
    neg_inner = 2.0 * s0 * s0 - jnp.sum(s * s, axis=1, keepdims=True)
    return s / jnp.sqrt(jnp.maximum(jnp.abs(neg_inner), 1e-8))


def _tc1_body(x_ref, w1s_ref, w1t_ref, b1_ref, p_ref, out_ref):
    x = x_ref[...]
    sq = jnp.sum(x * x, axis=1, keepdims=True)
    nrm = jnp.sqrt(jnp.maximum(sq, 1e-8))
    e = jnp.exp(nrm)
    shn = 0.5 * (e - 1.0 / e)
    xs = x * (shn / nrm)
    time = jnp.sqrt(1.0 + jnp.sum(xs * xs, axis=1, keepdims=True))
    v = (jnp.dot(xs, w1s_ref[...], preferred_element_type=jnp.float32,
                 precision=lax.Precision.HIGHEST)
         + time * w1t_ref[...] + b1_ref[...])
    out_ref[...] = _pack_bf16(_reparam(v, p_ref[0, 0]))


def _tc2_body(lo_ref, hi_ref, w2_ref, b2_ref, p_ref, h_ref, y_ref):
    s = lo_ref[...] + hi_ref[...]
    h = _centroid(s)
    h_ref[...] = h
    xr = jnp.maximum(h, 0.0)
    v = jnp.dot(xr, w2_ref[...], preferred_element_type=jnp.float32,
                precision=lax.Precision.HIGHEST) + b2_ref[...]
    y_ref[...] = _pack_bf16(_reparam(v, p_ref[0, 1]))


def _tc3_body(lo_ref, hi_ref, h_ref, p_ref, out_ref):
    y = _centroid(lo_ref[...] + hi_ref[...])
    z = p_ref[0, 2] * h_ref[...] + p_ref[0, 3] * y
    out_ref[...] = _centroid(z)


def _row_spec():
    return pl.BlockSpec((_B, _D), lambda i: (i, 0))


def _hi_spec():
    return pl.BlockSpec((_B, _D), lambda i: (i + _N // _B, 0))


def _w_spec(shape):
    return pl.BlockSpec(shape, lambda i: (0, 0))


def _p_spec():
    return pl.BlockSpec(memory_space=pltpu.SMEM)


_GRID = (_N // _B,)


def _pk_spec():
    return pl.BlockSpec((_B, _D // 2), lambda i: (i, 0))


def _tc1(x, w1s, w1t, b1, params):
    return pl.pallas_call(
        _tc1_body,
        grid=_GRID,
        in_specs=[_row_spec(), _w_spec((_D, _D)), _w_spec((1, _D)),
                  _w_spec((1, _D)), _p_spec()],
        out_specs=_pk_spec(),
        out_shape=jax.ShapeDtypeStruct((_N, _D // 2), jnp.int32),
    )(x, w1s, w1t, b1, params)


def _tc2(partials, w2, b2, params):
    return pl.pallas_call(
        _tc2_body,
        grid=_GRID,
        in_specs=[_row_spec(), _hi_spec(), _w_spec((_D, _D)),
                  _w_spec((1, _D)), _p_spec()],
        out_specs=[_row_spec(), _pk_spec()],
        out_shape=[jax.ShapeDtypeStruct((_N, _D), jnp.float32),
                   jax.ShapeDtypeStruct((_N, _D // 2), jnp.int32)],
    )(partials, partials, w2, b2, params)


def _tc3(partials, h, params):
    return pl.pallas_call(
        _tc3_body,
        grid=_GRID,
        in_specs=[_row_spec(), _hi_spec(), _row_spec(), _p_spec()],
        out_specs=_row_spec(),
        out_shape=jax.ShapeDtypeStruct((_N, _D), jnp.float32),
    )(partials, partials, h, params)


def kernel(x, adj, W1, b1, scale1, W2, b2, scale2, wx, wy):
    npad = _EPAD - _E
    src2d = jnp.concatenate(
        [adj[0], jnp.zeros((npad,), jnp.int32)]).reshape(
            _NSC * _NTILE, _NCH, _C)
    dst2d = jnp.concatenate(
        [adj[1], _N + (jnp.arange(npad, dtype=jnp.int32) % _NPAD)]).reshape(
            _NSC * _NTILE, _NCH, _C)
    zeros = jnp.zeros((_N + _NPAD, _D), dtype=jnp.float32)
    params = jnp.stack([jnp.exp(scale1), jnp.exp(scale2), wx, wy]).reshape(1, 4)
    b1r = b1.reshape(1, _D)
    b2r = b2.reshape(1, _D)
    w1t = W1[0:1, :]
    w1s = W1[1:, :]

    h1_packed = _tc1(x, w1s, w1t, b1r, params)
    partials1 = _sc_segment_sum(h1_packed, src2d, dst2d, zeros)
    h, y_packed = _tc2(partials1, W2, b2r, params)
    partials2 = _sc_segment_sum(y_packed, src2d, dst2d, zeros)
    return _tc3(partials2, h, params)


# speedup vs baseline: 1.5541x; 1.2164x over previous
import functools

import jax
import jax.numpy as jnp
from jax import lax
from jax.experimental import pallas as pl
from jax.experimental.pallas import tpu as pltpu
from jax.experimental.pallas import tpu_sc as plsc

_N = 10000
_D = 128
_E = 320000
_NSC = 2
_NTILE = 16
_C = 80
_NCH = 125
_WR_TILES = 10
_ROWS_OUT = _N // _WR_TILES


def _sc_segment_sum(h, src2d, dst2d, zeros):
    mesh = plsc.VectorSubcoreMesh(
        core_axis_name="c", subcore_axis_name="s",
        num_cores=_NSC, num_subcores=_NTILE)

    @functools.partial(
        pl.kernel,
        out_type=jax.ShapeDtypeStruct((_NSC * _N, _D), jnp.float32),
        mesh=mesh,
        scratch_types=[
            pltpu.VMEM((1, _NCH, _C), jnp.int32),
            pltpu.VMEM((1, _NCH, _C), jnp.int32),
            pltpu.VMEM((_C, _D // 2), jnp.int32),
            pltpu.VMEM((_C, _D // 2), jnp.int32),
            pltpu.VMEM((_C, _D), jnp.float32),
            pltpu.VMEM((_C, _D), jnp.float32),
            pltpu.VMEM_SHARED((_N, _D), jnp.float32),
            pltpu.SemaphoreType.DMA,
            pltpu.SemaphoreType.DMA,
            pltpu.SemaphoreType.DMA,
            pltpu.SemaphoreType.DMA,
        ],
        compiler_params=pltpu.CompilerParams(use_tc_tiling_on_sc=False,
                                             needs_layout_passes=False),
    )
    def run(h_hbm, src_hbm, dst_hbm, z_hbm, out_hbm,
            src_v, dst_v, rows0, rows1, conv0, conv1, acc,
            semg0, semg1, sems0, sems1):
        cid = lax.axis_index("c")
        sid = lax.axis_index("s")

        @pl.when(sid == 0)
        def _zero():
            pltpu.sync_copy(z_hbm, acc)

        tid = cid * _NTILE + sid
        pltpu.sync_copy(src_hbm.at[pl.ds(tid, 1)], src_v)
        pltpu.sync_copy(dst_hbm.at[pl.ds(tid, 1)], dst_v)
        plsc.subcore_barrier()

        rows = (rows0, rows1)
        convs = (conv0, conv1)
        semg = (semg0, semg1)
        sems = (sems0, sems1)

        pltpu.async_copy(h_hbm.at[src_v.at[0, 0]], rows0, semg0)
        pltpu.async_copy(h_hbm.at[src_v.at[0, 1]], rows1, semg1)

        def step(j, b):
            pltpu.make_async_copy(h_hbm.at[src_v.at[0, j]],
                                  rows[b], semg[b]).wait()

            @pl.when(j >= 2)
            def _drain():
                pltpu.make_async_copy(convs[b], acc.at[dst_v.at[0, 0]],
                                      sems[b]).wait()

            def crow(r, c):
                buf = rows[b]
                for g in range(4):
                    w = buf[r, pl.ds(16 * g, 16)]
                    convs[b][r, pl.ds(32 * g, 16)] = plsc.bitcast(
                        w << 16, jnp.float32)
                    convs[b][r, pl.ds(32 * g + 16, 16)] = plsc.bitcast(
                        w & jnp.int32(-65536), jnp.float32)
                return c

            lax.fori_loop(0, _C, crow, 0)
            pltpu.async_copy(convs[b], acc.at[dst_v.at[0, j]], sems[b],
                             add=True)

            @pl.when(j + 2 < _NCH)
            def _next():
                pltpu.async_copy(h_hbm.at[src_v.at[0, j + 2]],
                                 rows[b], semg[b])

        def dbl(i, c):
            step(i * 2, 0)
            step(i * 2 + 1, 1)
            return c

        lax.fori_loop(0, _NCH // 2, dbl, 0)
        if _NCH % 2:
            step(_NCH - 1, 0)
        for b in range(2):
            pltpu.make_async_copy(convs[b], acc.at[dst_v.at[0, 0]],
                                  sems[b]).wait()
        plsc.subcore_barrier()

        @pl.when(sid < _WR_TILES)
        def _writeout():
            rbase = sid * _ROWS_OUT
            pltpu.sync_copy(acc.at[pl.ds(rbase, _ROWS_OUT)],
                            out_hbm.at[pl.ds(cid * _N + rbase, _ROWS_OUT)])

    return run(h, src2d, dst2d, zeros)


_B = 1000


def _pack_bf16(v):
    a = v.astype(jnp.bfloat16)
    parts = []
    for g in range(4):
        lo = lax.convert_element_type(
            lax.bitcast_convert_type(a[:, 32 * g:32 * g + 16], jnp.uint16),
            jnp.int32)
        hi = lax.convert_element_type(
            lax.bitcast_convert_type(a[:, 32 * g + 16:32 * g + 32], jnp.uint16),
            jnp.int32)
        parts.append((hi << 16) | lo)
    return jnp.concatenate(parts, axis=1)


def _reparam(v, escale):
    v0 = v[:, 0:1]
    time = escale / (1.0 + jnp.exp(-v0)) + 1.1
    sq = jnp.maximum(jnp.sum(v * v, axis=1, keepdims=True) - v0 * v0, 1e-8)
    sc = (time * time - 1.0) / sq
    scaled = v * jnp.sqrt(sc)
    col = lax.broadcasted_iota(jnp.int32, v.shape, 1)
    return jnp.where(col == 0, time, scaled)


def _centroid(s):
    s0 = s[:, 0:1]
    neg_inner = 2.0 * s0 * s0 - jnp.sum(s * s, axis=1, keepdims=True)
    return s / jnp.sqrt(jnp.maximum(jnp.abs(neg_inner), 1e-8))


def _tc1_body(x_ref, w1s_ref, w1t_ref, b1_ref, p_ref, out_ref):
    x = x_ref[...]
    sq = jnp.sum(x * x, axis=1, keepdims=True)
    nrm = jnp.sqrt(jnp.maximum(sq, 1e-8))
    e = jnp.exp(nrm)
    shn = 0.5 * (e - 1.0 / e)
    xs = x * (shn / nrm)
    time = jnp.sqrt(1.0 + jnp.sum(xs * xs, axis=1, keepdims=True))
    v = (jnp.dot(xs, w1s_ref[...], preferred_element_type=jnp.float32,
                 precision=lax.Precision.HIGHEST)
         + time * w1t_ref[...] + b1_ref[...])
    out_ref[...] = _pack_bf16(_reparam(v, p_ref[0, 0]))


def _tc2_body(lo_ref, hi_ref, w2_ref, b2_ref, p_ref, h_ref, y_ref):
    s = lo_ref[...] + hi_ref[...]
    h = _centroid(s)
    h_ref[...] = h
    xr = jnp.maximum(h, 0.0)
    v = jnp.dot(xr, w2_ref[...], preferred_element_type=jnp.float32,
                precision=lax.Precision.HIGHEST) + b2_ref[...]
    y_ref[...] = _pack_bf16(_reparam(v, p_ref[0, 1]))


def _tc3_body(lo_ref, hi_ref, h_ref, p_ref, out_ref):
    y = _centroid(lo_ref[...] + hi_ref[...])
    z = p_ref[0, 2] * h_ref[...] + p_ref[0, 3] * y
    out_ref[...] = _centroid(z)


def _row_spec():
    return pl.BlockSpec((_B, _D), lambda i: (i, 0))


def _hi_spec():
    return pl.BlockSpec((_B, _D), lambda i: (i + _N // _B, 0))


def _w_spec(shape):
    return pl.BlockSpec(shape, lambda i: (0, 0))


def _p_spec():
    return pl.BlockSpec(memory_space=pltpu.SMEM)


_GRID = (_N // _B,)


def _pk_spec():
    return pl.BlockSpec((_B, _D // 2), lambda i: (i, 0))


def _tc1(x, w1s, w1t, b1, params):
    return pl.pallas_call(
        _tc1_body,
        grid=_GRID,
        in_specs=[_row_spec(), _w_spec((_D, _D)), _w_spec((1, _D)),
                  _w_spec((1, _D)), _p_spec()],
        out_specs=_pk_spec(),
        out_shape=jax.ShapeDtypeStruct((_N, _D // 2), jnp.int32),
    )(x, w1s, w1t, b1, params)


def _tc2(partials, w2, b2, params):
    return pl.pallas_call(
        _tc2_body,
        grid=_GRID,
        in_specs=[_row_spec(), _hi_spec(), _w_spec((_D, _D)),
                  _w_spec((1, _D)), _p_spec()],
        out_specs=[_row_spec(), _pk_spec()],
        out_shape=[jax.ShapeDtypeStruct((_N, _D), jnp.float32),
                   jax.ShapeDtypeStruct((_N, _D // 2), jnp.int32)],
    )(partials, partials, w2, b2, params)


def _tc3(partials, h, params):
    return pl.pallas_call(
        _tc3_body,
        grid=_GRID,
        in_specs=[_row_spec(), _hi_spec(), _row_spec(), _p_spec()],
        out_specs=_row_spec(),
        out_shape=jax.ShapeDtypeStruct((_N, _D), jnp.float32),
    )(partials, partials, h, params)


def kernel(x, adj, W1, b1, scale1, W2, b2, scale2, wx, wy):
    src2d = adj[0].reshape(_NSC * _NTILE, _NCH, _C)
    dst2d = adj[1].reshape(_NSC * _NTILE, _NCH, _C)
    zeros = jnp.zeros((_N, _D), dtype=jnp.float32)
    params = jnp.stack([jnp.exp(scale1), jnp.exp(scale2), wx, wy]).reshape(1, 4)
    b1r = b1.reshape(1, _D)
    b2r = b2.reshape(1, _D)
    w1t = W1[0:1, :]
    w1s = W1[1:, :]

    h1_packed = _tc1(x, w1s, w1t, b1r, params)
    partials1 = _sc_segment_sum(h1_packed, src2d, dst2d, zeros)
    h, y_packed = _tc2(partials1, W2, b2r, params)
    partials2 = _sc_segment_sum(y_packed, src2d, dst2d, zeros)
    return _tc3(partials2, h, params)
